# baseline (speedup 1.0000x reference)
import functools

import jax
import jax.numpy as jnp
from jax import lax
from jax.experimental import pallas as pl
from jax.experimental.pallas import tpu as pltpu
from jax.experimental.pallas import tpu_sc as plsc

_W = 128
_R = 512


def _decode_threshold(cand):
    pos = cand >= jnp.uint32(0x80000000)
    b = jnp.where(pos, cand & jnp.uint32(0x7FFFFFFF), ~cand)
    f = jax.lax.bitcast_convert_type(b, jnp.float32)
    return jnp.where(cand >= jnp.uint32(0xFF800000), jnp.float32(jnp.inf), f)


def _median_shift_kernel(x_hbm, med_ref, nt_ref, o_hbm,
                         buf, stage, in_sems, out_sem, *, rank):
    j = pl.program_id(0)
    ng = pl.num_programs(0)
    n = buf.shape[1]
    slot = jax.lax.rem(j, 2)

    def in_copy(jj):
        return pltpu.make_async_copy(
            x_hbm.at[:, pl.ds(jj * _W, _W)],
            buf.at[jax.lax.rem(jj, 2)],
            in_sems.at[jax.lax.rem(jj, 2)],
        )

    def out_copy(jj):
        return pltpu.make_async_copy(
            stage, o_hbm.at[:, pl.ds(jj * _W, _W)], out_sem)

    @pl.when(j == 0)
    def _():
        in_copy(j).start()

    @pl.when(j + 1 < ng)
    def _():
        in_copy(j + 1).start()

    in_copy(j).wait()

    kplus1 = jnp.int32(rank + 1)
    nchunks = n // _R

    def bit_body(i, prefix):
        bit = jnp.uint32(31) - i.astype(jnp.uint32)
        low_mask = (jnp.uint32(1) << bit) - jnp.uint32(1)
        cand = prefix | low_mask
        thr = _decode_threshold(cand)

        def chunk_body(r, acc8):
            ch = buf[slot, pl.ds(r * _R, _R), :]
            m = (ch <= thr).astype(jnp.int32).reshape(_R // 8, 8, _W)
            return acc8 + jnp.sum(m, axis=0)

        acc8 = jax.lax.fori_loop(
            0, nchunks, chunk_body, jnp.zeros((8, _W), jnp.int32))
        cnt = jnp.sum(acc8, axis=0, keepdims=True)
        return jnp.where(cnt >= kplus1, prefix,
                         prefix | (low_mask + jnp.uint32(1)))

    prefix0 = jnp.zeros((1, _W), dtype=jnp.uint32)
    sel = jax.lax.fori_loop(0, 32, bit_body, prefix0)
    med = _decode_threshold(sel)

    nt = nt_ref[0, 0]
    new_med = (med_ref[...] * nt + med) / (nt + jnp.float32(1.0))

    @pl.when(j >= 1)
    def _():
        out_copy(j - 1).wait()

    def sub_body(r, _):
        rows = pl.ds(r * 1024, 1024)
        stage[rows, :] = buf[slot, rows, :] - new_med
        return 0

    jax.lax.fori_loop(0, n // 1024, sub_body, 0)
    out_copy(j).start()

    @pl.when(j == ng - 1)
    def _():
        out_copy(j).wait()


def _sc_colsum(x):
    n = x.shape[0]
    rows_per_w = n // 32
    mesh = plsc.VectorSubcoreMesh(core_axis_name="c", subcore_axis_name="s")

    @functools.partial(
        pl.kernel,
        out_type=jax.ShapeDtypeStruct((32, 128), jnp.float32),
        mesh=mesh,
        scratch_types=[
            pltpu.VMEM((64, 128), jnp.float32),
            pltpu.VMEM((128,), jnp.float32),
        ],
    )
    def body(x_hbm, out_hbm, buf, acc):
        wid = lax.axis_index("s") * 2 + lax.axis_index("c")
        r0 = wid * rows_per_w
        for g in range(8):
            acc[pl.ds(g * 16, 16)] = jnp.zeros((16,), jnp.float32)

        def chunk(i, carry):
            pltpu.sync_copy(
                x_hbm.at[pl.ds(r0 + jax.lax.rem(i, rows_per_w // 64) * 64, 64),
                         pl.ds(640, 128)], buf)

            def row(r, cr):
                for g in range(8):
                    sl = pl.ds(g * 16, 16)
                    acc[sl] = acc[sl] + buf[r, sl]
                return cr

            return jax.lax.fori_loop(0, 64, row, carry)

        jax.lax.fori_loop(0, 10 * (rows_per_w // 64), chunk, 0)
        pltpu.sync_copy(acc, out_hbm.at[wid])

    return body(x)


def kernel(x, median, num_track):
    n, c = x.shape
    grid = (c // _W,)
    nt = num_track.astype(jnp.float32).reshape(1, 1)

    fn = functools.partial(_median_shift_kernel, rank=n // 2)
    probe = _sc_colsum(x)
    nt = nt + 0.0 * probe[0, 0]
    out = pl.pallas_call(
        fn,
        grid=grid,
        in_specs=[
            pl.BlockSpec(memory_space=pltpu.MemorySpace.HBM),
            pl.BlockSpec((1, _W), lambda j: (0, j)),
            pl.BlockSpec(memory_space=pltpu.SMEM),
        ],
        out_specs=pl.BlockSpec(memory_space=pltpu.MemorySpace.HBM),
        out_shape=jax.ShapeDtypeStruct((n, c), jnp.float32),
        scratch_shapes=[
            pltpu.VMEM((2, n, _W), jnp.float32),
            pltpu.VMEM((n, _W), jnp.float32),
            pltpu.SemaphoreType.DMA((2,)),
            pltpu.SemaphoreType.DMA,
        ],
        compiler_params=pltpu.CompilerParams(
            dimension_semantics=("arbitrary",)),
    )(x, median, nt)
    return out


# speedup vs baseline: 1.9547x; 1.9547x over previous
import functools

import jax
import jax.numpy as jnp
from jax.experimental import pallas as pl
from jax.experimental.pallas import tpu as pltpu

_W = 128
_R = 512


def _decode_threshold(cand):
    pos = cand >= jnp.uint32(0x80000000)
    b = jnp.where(pos, cand & jnp.uint32(0x7FFFFFFF), ~cand)
    f = jax.lax.bitcast_convert_type(b, jnp.float32)
    return jnp.where(cand >= jnp.uint32(0xFF800000), jnp.float32(jnp.inf), f)


def _encode_hi16(b32):
    t = b32 >> jnp.uint32(16)
    neg = b32 >= jnp.uint32(0x80000000)
    enc = jnp.where(neg, ~t & jnp.uint32(0xFFFF), t | jnp.uint32(0x8000))
    return (enc ^ jnp.uint32(0x8000)).astype(jnp.int32)


def _median_shift_kernel(x_hbm, med_ref, nt_ref, o_hbm,
                         buf, e16, stage, in_sems, out_sem, *, rank):
    j = pl.program_id(0)
    ng = pl.num_programs(0)
    n = stage.shape[0]
    slot = jax.lax.rem(j, 2)

    def in_copy(jj):
        return pltpu.make_async_copy(
            x_hbm.at[:, pl.ds(jj * _W, _W)],
            buf.at[jax.lax.rem(jj, 2)],
            in_sems.at[jax.lax.rem(jj, 2)],
        )

    def out_copy(jj):
        return pltpu.make_async_copy(
            stage, o_hbm.at[:, pl.ds(jj * _W, _W)], out_sem)

    @pl.when(j == 0)
    def _():
        in_copy(j).start()

    @pl.when(j + 1 < ng)
    def _():
        in_copy(j + 1).start()

    in_copy(j).wait()

    kplus1 = jnp.int32(rank + 1)
    nchunks = n // _R

    def enc_body(r, carry):
        rows = pl.ds(r * _R, _R)
        b32 = jax.lax.bitcast_convert_type(buf[slot, rows, :], jnp.uint32)
        e16[rows, :] = _encode_hi16(b32).astype(jnp.int16)
        return carry

    jax.lax.fori_loop(0, nchunks, enc_body, 0)

    def hi_body(i, prefix):
        bit = jnp.uint32(31) - i.astype(jnp.uint32)
        low_mask = (jnp.uint32(1) << bit) - jnp.uint32(1)
        cand = prefix | low_mask
        thr16 = ((cand >> jnp.uint32(16)) ^ jnp.uint32(0x8000)
                 ).astype(jnp.int32).astype(jnp.int16)

        def chunk_body(r, acc16):
            ch = e16[pl.ds(r * _R, _R), :]
            m = (ch <= thr16).astype(jnp.int16).reshape(_R // 16, 16, _W)
            parts = [m[q] for q in range(_R // 16)]
            while len(parts) > 1:
                nxt = [parts[q] + parts[q + 1] for q in range(0, len(parts) - 1, 2)]
                if len(parts) % 2:
                    nxt.append(parts[-1])
                parts = nxt
            return acc16 + parts[0]

        acc16 = jax.lax.fori_loop(
            0, nchunks, chunk_body, jnp.zeros((16, _W), jnp.int16))
        cnt = jnp.sum(acc16.astype(jnp.int32), axis=0, keepdims=True)
        return jnp.where(cnt >= kplus1, prefix,
                         prefix | (low_mask + jnp.uint32(1)))

    def lo_body(i, prefix):
        bit = jnp.uint32(31) - i.astype(jnp.uint32)
        low_mask = (jnp.uint32(1) << bit) - jnp.uint32(1)
        cand = prefix | low_mask
        thr = _decode_threshold(cand)

        def chunk_body(r, acc8):
            ch = buf[slot, pl.ds(r * _R, _R), :]
            m = (ch <= thr).astype(jnp.int32).reshape(_R // 8, 8, _W)
            return acc8 + jnp.sum(m, axis=0)

        acc8 = jax.lax.fori_loop(
            0, nchunks, chunk_body, jnp.zeros((8, _W), jnp.int32))
        cnt = jnp.sum(acc8, axis=0, keepdims=True)
        return jnp.where(cnt >= kplus1, prefix,
                         prefix | (low_mask + jnp.uint32(1)))

    prefix0 = jnp.zeros((1, _W), dtype=jnp.uint32)
    sel = jax.lax.fori_loop(0, 16, hi_body, prefix0)
    sel = jax.lax.fori_loop(16, 32, lo_body, sel)
    med = _decode_threshold(sel)

    nt = nt_ref[0, 0]
    new_med = (med_ref[...] * nt + med) / (nt + jnp.float32(1.0))

    @pl.when(j >= 1)
    def _():
        out_copy(j - 1).wait()

    def sub_body(r, _):
        rows = pl.ds(r * 1024, 1024)
        stage[rows, :] = buf[slot, rows, :] - new_med
        return 0

    jax.lax.fori_loop(0, n // 1024, sub_body, 0)
    out_copy(j).start()

    @pl.when(j == ng - 1)
    def _():
        out_copy(j).wait()


def kernel(x, median, num_track):
    n, c = x.shape
    grid = (c // _W,)
    nt = num_track.astype(jnp.float32).reshape(1, 1)

    fn = functools.partial(_median_shift_kernel, rank=n // 2)
    return pl.pallas_call(
        fn,
        grid=grid,
        in_specs=[
            pl.BlockSpec(memory_space=pltpu.MemorySpace.HBM),
            pl.BlockSpec((1, _W), lambda j: (0, j)),
            pl.BlockSpec(memory_space=pltpu.SMEM),
        ],
        out_specs=pl.BlockSpec(memory_space=pltpu.MemorySpace.HBM),
        out_shape=jax.ShapeDtypeStruct((n, c), jnp.float32),
        scratch_shapes=[
            pltpu.VMEM((2, n, _W), jnp.float32),
            pltpu.VMEM((n, _W), jnp.int16),
            pltpu.VMEM((n, _W), jnp.float32),
            pltpu.SemaphoreType.DMA((2,)),
            pltpu.SemaphoreType.DMA,
        ],
        compiler_params=pltpu.CompilerParams(
            dimension_semantics=("arbitrary",)),
    )(x, median, nt)


# speedup vs baseline: 1.9714x; 1.0086x over previous
import functools

import jax
import jax.numpy as jnp
from jax.experimental import pallas as pl
from jax.experimental.pallas import tpu as pltpu

_W = 128
_R = 512


def _decode_threshold(cand):
    pos = cand >= jnp.uint32(0x80000000)
    b = jnp.where(pos, cand & jnp.uint32(0x7FFFFFFF), ~cand)
    f = jax.lax.bitcast_convert_type(b, jnp.float32)
    return jnp.where(cand >= jnp.uint32(0xFF800000), jnp.float32(jnp.inf), f)


def _encode_hi16(b32):
    t = b32 >> jnp.uint32(16)
    neg = b32 >= jnp.uint32(0x80000000)
    enc = jnp.where(neg, ~t & jnp.uint32(0xFFFF), t | jnp.uint32(0x8000))
    return (enc ^ jnp.uint32(0x8000)).astype(jnp.int32)


def _median_shift_kernel(x_hbm, med_ref, nt_ref, o_hbm,
                         buf, e16, stage, in_sems, out_sem, *, rank):
    j = pl.program_id(0)
    ng = pl.num_programs(0)
    n = stage.shape[0]
    slot = jax.lax.rem(j, 2)

    def in_copy(jj):
        return pltpu.make_async_copy(
            x_hbm.at[:, pl.ds(jj * _W, _W)],
            buf.at[jax.lax.rem(jj, 2)],
            in_sems.at[jax.lax.rem(jj, 2)],
        )

    def out_copy(jj):
        return pltpu.make_async_copy(
            stage, o_hbm.at[:, pl.ds(jj * _W, _W)], out_sem)

    @pl.when(j == 0)
    def _():
        in_copy(j).start()

    @pl.when(j + 1 < ng)
    def _():
        in_copy(j + 1).start()

    in_copy(j).wait()

    kplus1 = jnp.int32(rank + 1)
    nchunks = n // _R

    def enc_body(r, acc8):
        rows = pl.ds(r * _R, _R)
        ch = buf[slot, rows, :]
        b32 = jax.lax.bitcast_convert_type(ch, jnp.uint32)
        e16[rows, :] = _encode_hi16(b32).astype(jnp.int16)
        m = (ch <= jnp.float32(-0.0)).astype(jnp.int32).reshape(_R // 8, 8, _W)
        return acc8 + jnp.sum(m, axis=0)

    acc8 = jax.lax.fori_loop(
        0, nchunks, enc_body, jnp.zeros((8, _W), jnp.int32))
    cnt0 = jnp.sum(acc8, axis=0, keepdims=True)
    prefix0 = jnp.where(cnt0 >= kplus1,
                        jnp.zeros((1, _W), jnp.uint32),
                        jnp.full((1, _W), 0x80000000, jnp.uint32))

    _RH = 2 * _R
    nhchunks = n // _RH

    def hi_body(i, prefix):
        bit = jnp.uint32(31) - i.astype(jnp.uint32)
        low_mask = (jnp.uint32(1) << bit) - jnp.uint32(1)
        cand = prefix | low_mask
        thr16 = ((cand >> jnp.uint32(16)) ^ jnp.uint32(0x8000)
                 ).astype(jnp.int32).astype(jnp.int16)

        def chunk_body(r, acc16):
            ch = e16[pl.ds(r * _RH, _RH), :]
            m = (ch <= thr16).astype(jnp.int16).reshape(_RH // 16, 16, _W)
            parts = [m[q] for q in range(_RH // 16)]
            while len(parts) > 1:
                nxt = [parts[q] + parts[q + 1] for q in range(0, len(parts) - 1, 2)]
                if len(parts) % 2:
                    nxt.append(parts[-1])
                parts = nxt
            return acc16 + parts[0]

        acc16 = jax.lax.fori_loop(
            0, nhchunks, chunk_body, jnp.zeros((16, _W), jnp.int16))
        cnt = jnp.sum(acc16.astype(jnp.int32), axis=0, keepdims=True)
        return jnp.where(cnt >= kplus1, prefix,
                         prefix | (low_mask + jnp.uint32(1)))

    def lo_body(i, prefix):
        bit = jnp.uint32(31) - i.astype(jnp.uint32)
        low_mask = (jnp.uint32(1) << bit) - jnp.uint32(1)
        cand = prefix | low_mask
        thr = _decode_threshold(cand)

        def chunk_body(r, acc8):
            ch = buf[slot, pl.ds(r * _R, _R), :]
            m = (ch <= thr).astype(jnp.int32).reshape(_R // 8, 8, _W)
            return acc8 + jnp.sum(m, axis=0)

        acc8 = jax.lax.fori_loop(
            0, nchunks, chunk_body, jnp.zeros((8, _W), jnp.int32))
        cnt = jnp.sum(acc8, axis=0, keepdims=True)
        return jnp.where(cnt >= kplus1, prefix,
                         prefix | (low_mask + jnp.uint32(1)))

    prefix0 = jnp.zeros((1, _W), dtype=jnp.uint32)
    sel = jax.lax.fori_loop(0, 16, hi_body, prefix0)
    sel = jax.lax.fori_loop(16, 32, lo_body, sel)
    med = _decode_threshold(sel)

    nt = nt_ref[0, 0]
    new_med = (med_ref[...] * nt + med) / (nt + jnp.float32(1.0))

    @pl.when(j >= 1)
    def _():
        out_copy(j - 1).wait()

    def sub_body(r, _):
        rows = pl.ds(r * 1024, 1024)
        stage[rows, :] = buf[slot, rows, :] - new_med
        return 0

    jax.lax.fori_loop(0, n // 1024, sub_body, 0)
    out_copy(j).start()

    @pl.when(j == ng - 1)
    def _():
        out_copy(j).wait()


def kernel(x, median, num_track):
    n, c = x.shape
    grid = (c // _W,)
    nt = num_track.astype(jnp.float32).reshape(1, 1)

    fn = functools.partial(_median_shift_kernel, rank=n // 2)
    return pl.pallas_call(
        fn,
        grid=grid,
        in_specs=[
            pl.BlockSpec(memory_space=pltpu.MemorySpace.HBM),
            pl.BlockSpec((1, _W), lambda j: (0, j)),
            pl.BlockSpec(memory_space=pltpu.SMEM),
        ],
        out_specs=pl.BlockSpec(memory_space=pltpu.MemorySpace.HBM),
        out_shape=jax.ShapeDtypeStruct((n, c), jnp.float32),
        scratch_shapes=[
            pltpu.VMEM((2, n, _W), jnp.float32),
            pltpu.VMEM((n, _W), jnp.int16),
            pltpu.VMEM((n, _W), jnp.float32),
            pltpu.SemaphoreType.DMA((2,)),
            pltpu.SemaphoreType.DMA,
        ],
        compiler_params=pltpu.CompilerParams(
            dimension_semantics=("arbitrary",)),
    )(x, median, nt)


# speedup vs baseline: 2.5880x; 1.3128x over previous
import functools

import jax
import jax.numpy as jnp
from jax.experimental import pallas as pl
from jax.experimental.pallas import tpu as pltpu

_W = 128
_R = 512


def _decode_threshold(cand):
    pos = cand >= jnp.uint32(0x80000000)
    b = jnp.where(pos, cand & jnp.uint32(0x7FFFFFFF), ~cand)
    f = jax.lax.bitcast_convert_type(b, jnp.float32)
    return jnp.where(cand >= jnp.uint32(0xFF800000), jnp.float32(jnp.inf), f)


def _median_shift_kernel(x_hbm, med_ref, nt_ref, o_hbm,
                         buf, stage, in_sems, out_sem, *, rank):
    j = pl.program_id(0)
    ng = pl.num_programs(0)
    n = buf.shape[1]
    slot = jax.lax.rem(j, 2)

    def in_copy(jj):
        return pltpu.make_async_copy(
            x_hbm.at[:, pl.ds(jj * _W, _W)],
            buf.at[jax.lax.rem(jj, 2)],
            in_sems.at[jax.lax.rem(jj, 2)],
        )

    def out_copy(jj):
        return pltpu.make_async_copy(
            stage, o_hbm.at[:, pl.ds(jj * _W, _W)], out_sem)

    @pl.when(j == 0)
    def _():
        in_copy(j).start()

    @pl.when(j + 1 < ng)
    def _():
        in_copy(j + 1).start()

    in_copy(j).wait()

    kplus1 = jnp.int32(rank + 1)
    nchunks = n // _R

    def bit_body(i, prefix):
        bit = jnp.uint32(31) - i.astype(jnp.uint32)
        low_mask = (jnp.uint32(1) << bit) - jnp.uint32(1)
        cand = prefix | low_mask
        thr = _decode_threshold(cand)

        def chunk_body(r, accs):
            ch = buf[slot, pl.ds(r * _R, _R), :]
            m = (ch <= thr).reshape(_R // 8, 8, _W)
            accs = list(accs)
            for q in range(_R // 8):
                a = accs[q % 4]
                accs[q % 4] = jnp.where(m[q], a + 1, a)
            return tuple(accs)

        zero8 = jnp.zeros((8, _W), jnp.int32)
        accs = jax.lax.fori_loop(
            0, nchunks, chunk_body, (zero8, zero8, zero8, zero8))
        acc8 = (accs[0] + accs[1]) + (accs[2] + accs[3])
        cnt = jnp.sum(acc8, axis=0, keepdims=True)
        return jnp.where(cnt >= kplus1, prefix,
                         prefix | (low_mask + jnp.uint32(1)))

    prefix0 = jnp.zeros((1, _W), dtype=jnp.uint32)
    sel = jax.lax.fori_loop(0, 32, bit_body, prefix0)
    med = _decode_threshold(sel)

    nt = nt_ref[0, 0]
    new_med = (med_ref[...] * nt + med) / (nt + jnp.float32(1.0))

    @pl.when(j >= 1)
    def _():
        out_copy(j - 1).wait()

    def sub_body(r, _):
        rows = pl.ds(r * 1024, 1024)
        stage[rows, :] = buf[slot, rows, :] - new_med
        return 0

    jax.lax.fori_loop(0, n // 1024, sub_body, 0)
    out_copy(j).start()

    @pl.when(j == ng - 1)
    def _():
        out_copy(j).wait()


def kernel(x, median, num_track):
    n, c = x.shape
    grid = (c // _W,)
    nt = num_track.astype(jnp.float32).reshape(1, 1)

    fn = functools.partial(_median_shift_kernel, rank=n // 2)
    return pl.pallas_call(
        fn,
        grid=grid,
        in_specs=[
            pl.BlockSpec(memory_space=pltpu.MemorySpace.HBM),
            pl.BlockSpec((1, _W), lambda j: (0, j)),
            pl.BlockSpec(memory_space=pltpu.SMEM),
        ],
        out_specs=pl.BlockSpec(memory_space=pltpu.MemorySpace.HBM),
        out_shape=jax.ShapeDtypeStruct((n, c), jnp.float32),
        scratch_shapes=[
            pltpu.VMEM((2, n, _W), jnp.float32),
            pltpu.VMEM((n, _W), jnp.float32),
            pltpu.SemaphoreType.DMA((2,)),
            pltpu.SemaphoreType.DMA,
        ],
        compiler_params=pltpu.CompilerParams(
            dimension_semantics=("arbitrary",)),
    )(x, median, nt)


# speedup vs baseline: 2.7785x; 1.0736x over previous
import functools

import jax
import jax.numpy as jnp
from jax.experimental import pallas as pl
from jax.experimental.pallas import tpu as pltpu

_W = 128
_R = 1024


def _decode_threshold(cand):
    pos = cand >= jnp.uint32(0x80000000)
    b = jnp.where(pos, cand & jnp.uint32(0x7FFFFFFF), ~cand)
    f = jax.lax.bitcast_convert_type(b, jnp.float32)
    return jnp.where(cand >= jnp.uint32(0xFF800000), jnp.float32(jnp.inf), f)


def _median_shift_kernel(x_hbm, med_ref, nt_ref, o_hbm,
                         buf, stage, in_sems, out_sem, *, rank):
    j = pl.program_id(0)
    ng = pl.num_programs(0)
    n = buf.shape[1]
    slot = jax.lax.rem(j, 2)

    def in_copy(jj):
        return pltpu.make_async_copy(
            x_hbm.at[:, pl.ds(jj * _W, _W)],
            buf.at[jax.lax.rem(jj, 2)],
            in_sems.at[jax.lax.rem(jj, 2)],
        )

    def out_copy(jj):
        return pltpu.make_async_copy(
            stage, o_hbm.at[:, pl.ds(jj * _W, _W)], out_sem)

    @pl.when(j == 0)
    def _():
        in_copy(j).start()

    @pl.when(j + 1 < ng)
    def _():
        in_copy(j + 1).start()

    in_copy(j).wait()

    kplus1 = jnp.int32(rank + 1)
    nchunks = n // _R

    def bit_body(i, prefix):
        bit = jnp.uint32(31) - i.astype(jnp.uint32)
        low_mask = (jnp.uint32(1) << bit) - jnp.uint32(1)
        cand = prefix | low_mask
        thr = _decode_threshold(cand)

        def chunk_body(r, accs):
            ch = buf[slot, pl.ds(r * _R, _R), :]
            m = (ch <= thr).reshape(_R // 8, 8, _W)
            accs = list(accs)
            for q in range(_R // 8):
                a = accs[q % 8]
                accs[q % 8] = jnp.where(m[q], a + 1, a)
            return tuple(accs)

        zero8 = jnp.zeros((8, _W), jnp.int32)
        accs = jax.lax.fori_loop(
            0, nchunks, chunk_body, tuple([zero8] * 8))
        acc8 = ((accs[0] + accs[1]) + (accs[2] + accs[3])) + ((accs[4] + accs[5]) + (accs[6] + accs[7]))
        cnt = jnp.sum(acc8, axis=0, keepdims=True)
        return jnp.where(cnt >= kplus1, prefix,
                         prefix | (low_mask + jnp.uint32(1)))

    prefix0 = jnp.zeros((1, _W), dtype=jnp.uint32)
    sel = jax.lax.fori_loop(0, 32, bit_body, prefix0)
    med = _decode_threshold(sel)

    nt = nt_ref[0, 0]
    new_med = (med_ref[...] * nt + med) / (nt + jnp.float32(1.0))

    @pl.when(j >= 1)
    def _():
        out_copy(j - 1).wait()

    def sub_body(r, _):
        rows = pl.ds(r * 1024, 1024)
        stage[rows, :] = buf[slot, rows, :] - new_med
        return 0

    jax.lax.fori_loop(0, n // 1024, sub_body, 0)
    out_copy(j).start()

    @pl.when(j == ng - 1)
    def _():
        out_copy(j).wait()


def kernel(x, median, num_track):
    n, c = x.shape
    grid = (c // _W,)
    nt = num_track.astype(jnp.float32).reshape(1, 1)

    fn = functools.partial(_median_shift_kernel, rank=n // 2)
    return pl.pallas_call(
        fn,
        grid=grid,
        in_specs=[
            pl.BlockSpec(memory_space=pltpu.MemorySpace.HBM),
            pl.BlockSpec((1, _W), lambda j: (0, j)),
            pl.BlockSpec(memory_space=pltpu.SMEM),
        ],
        out_specs=pl.BlockSpec(memory_space=pltpu.MemorySpace.HBM),
        out_shape=jax.ShapeDtypeStruct((n, c), jnp.float32),
        scratch_shapes=[
            pltpu.VMEM((2, n, _W), jnp.float32),
            pltpu.VMEM((n, _W), jnp.float32),
            pltpu.SemaphoreType.DMA((2,)),
            pltpu.SemaphoreType.DMA,
        ],
        compiler_params=pltpu.CompilerParams(
            dimension_semantics=("arbitrary",)),
    )(x, median, nt)


# speedup vs baseline: 2.8538x; 1.0271x over previous
import functools

import jax
import jax.numpy as jnp
from jax.experimental import pallas as pl
from jax.experimental.pallas import tpu as pltpu

_W = 128
_R = 2048


def _decode_threshold(cand):
    pos = cand >= jnp.uint32(0x80000000)
    b = jnp.where(pos, cand & jnp.uint32(0x7FFFFFFF), ~cand)
    f = jax.lax.bitcast_convert_type(b, jnp.float32)
    return jnp.where(cand >= jnp.uint32(0xFF800000), jnp.float32(jnp.inf), f)


def _median_shift_kernel(x_hbm, med_ref, nt_ref, o_hbm,
                         buf, stage, in_sems, out_sem, *, rank):
    j = pl.program_id(0)
    ng = pl.num_programs(0)
    n = buf.shape[1]
    slot = jax.lax.rem(j, 2)

    def in_copy(jj):
        return pltpu.make_async_copy(
            x_hbm.at[:, pl.ds(jj * _W, _W)],
            buf.at[jax.lax.rem(jj, 2)],
            in_sems.at[jax.lax.rem(jj, 2)],
        )

    def out_copy(jj):
        return pltpu.make_async_copy(
            stage, o_hbm.at[:, pl.ds(jj * _W, _W)], out_sem)

    @pl.when(j == 0)
    def _():
        in_copy(j).start()

    @pl.when(j + 1 < ng)
    def _():
        in_copy(j + 1).start()

    in_copy(j).wait()

    kplus1 = jnp.int32(rank + 1)
    nchunks = n // _R

    def bit_body(i, prefix):
        bit = jnp.uint32(31) - i.astype(jnp.uint32)
        low_mask = (jnp.uint32(1) << bit) - jnp.uint32(1)
        cand = prefix | low_mask
        thr = _decode_threshold(cand)

        def chunk_body(r, accs):
            ch = buf[slot, pl.ds(r * _R, _R), :]
            m = (ch <= thr).reshape(_R // 8, 8, _W)
            accs = list(accs)
            for q in range(_R // 8):
                a = accs[q % 8]
                accs[q % 8] = jnp.where(m[q], a + 1, a)
            return tuple(accs)

        zero8 = jnp.zeros((8, _W), jnp.int32)
        accs = jax.lax.fori_loop(
            0, nchunks, chunk_body, tuple([zero8] * 8))
        acc8 = ((accs[0] + accs[1]) + (accs[2] + accs[3])) + ((accs[4] + accs[5]) + (accs[6] + accs[7]))
        cnt = jnp.sum(acc8, axis=0, keepdims=True)
        return jnp.where(cnt >= kplus1, prefix,
                         prefix | (low_mask + jnp.uint32(1)))

    prefix0 = jnp.zeros((1, _W), dtype=jnp.uint32)
    sel = jax.lax.fori_loop(0, 32, bit_body, prefix0)
    med = _decode_threshold(sel)

    nt = nt_ref[0, 0]
    new_med = (med_ref[...] * nt + med) / (nt + jnp.float32(1.0))

    @pl.when(j >= 1)
    def _():
        out_copy(j - 1).wait()

    def sub_body(r, _):
        rows = pl.ds(r * 1024, 1024)
        stage[rows, :] = buf[slot, rows, :] - new_med
        return 0

    jax.lax.fori_loop(0, n // 1024, sub_body, 0)
    out_copy(j).start()

    @pl.when(j == ng - 1)
    def _():
        out_copy(j).wait()


def kernel(x, median, num_track):
    n, c = x.shape
    grid = (c // _W,)
    nt = num_track.astype(jnp.float32).reshape(1, 1)

    fn = functools.partial(_median_shift_kernel, rank=n // 2)
    return pl.pallas_call(
        fn,
        grid=grid,
        in_specs=[
            pl.BlockSpec(memory_space=pltpu.MemorySpace.HBM),
            pl.BlockSpec((1, _W), lambda j: (0, j)),
            pl.BlockSpec(memory_space=pltpu.SMEM),
        ],
        out_specs=pl.BlockSpec(memory_space=pltpu.MemorySpace.HBM),
        out_shape=jax.ShapeDtypeStruct((n, c), jnp.float32),
        scratch_shapes=[
            pltpu.VMEM((2, n, _W), jnp.float32),
            pltpu.VMEM((n, _W), jnp.float32),
            pltpu.SemaphoreType.DMA((2,)),
            pltpu.SemaphoreType.DMA,
        ],
        compiler_params=pltpu.CompilerParams(
            dimension_semantics=("arbitrary",)),
    )(x, median, nt)


# speedup vs baseline: 2.8921x; 1.0134x over previous
import functools

import jax
import jax.numpy as jnp
from jax.experimental import pallas as pl
from jax.experimental.pallas import tpu as pltpu

_W = 128
_R = 4096


def _decode_threshold(cand):
    pos = cand >= jnp.uint32(0x80000000)
    b = jnp.where(pos, cand & jnp.uint32(0x7FFFFFFF), ~cand)
    f = jax.lax.bitcast_convert_type(b, jnp.float32)
    return jnp.where(cand >= jnp.uint32(0xFF800000), jnp.float32(jnp.inf), f)


def _median_shift_kernel(x_hbm, med_ref, nt_ref, o_hbm,
                         buf, stage, in_sems, out_sem, *, rank):
    j = pl.program_id(0)
    ng = pl.num_programs(0)
    n = buf.shape[1]
    slot = jax.lax.rem(j, 2)

    def in_copy(jj):
        return pltpu.make_async_copy(
            x_hbm.at[:, pl.ds(jj * _W, _W)],
            buf.at[jax.lax.rem(jj, 2)],
            in_sems.at[jax.lax.rem(jj, 2)],
        )

    def out_copy(jj):
        return pltpu.make_async_copy(
            stage, o_hbm.at[:, pl.ds(jj * _W, _W)], out_sem)

    @pl.when(j == 0)
    def _():
        in_copy(j).start()

    @pl.when(j + 1 < ng)
    def _():
        in_copy(j + 1).start()

    in_copy(j).wait()

    kplus1 = jnp.int32(rank + 1)
    nchunks = n // _R

    def bit_body(i, prefix):
        bit = jnp.uint32(31) - i.astype(jnp.uint32)
        low_mask = (jnp.uint32(1) << bit) - jnp.uint32(1)
        cand = prefix | low_mask
        thr = _decode_threshold(cand)

        def chunk_body(r, accs):
            ch = buf[slot, pl.ds(r * _R, _R), :]
            m = (ch <= thr).reshape(_R // 8, 8, _W)
            accs = list(accs)
            for q in range(_R // 8):
                a = accs[q % 8]
                accs[q % 8] = jnp.where(m[q], a + 1, a)
            return tuple(accs)

        zero8 = jnp.zeros((8, _W), jnp.int32)
        accs = jax.lax.fori_loop(
            0, nchunks, chunk_body, tuple([zero8] * 8))
        acc8 = ((accs[0] + accs[1]) + (accs[2] + accs[3])) + ((accs[4] + accs[5]) + (accs[6] + accs[7]))
        cnt = jnp.sum(acc8, axis=0, keepdims=True)
        return jnp.where(cnt >= kplus1, prefix,
                         prefix | (low_mask + jnp.uint32(1)))

    prefix0 = jnp.zeros((1, _W), dtype=jnp.uint32)
    sel = jax.lax.fori_loop(0, 32, bit_body, prefix0)
    med = _decode_threshold(sel)

    nt = nt_ref[0, 0]
    new_med = (med_ref[...] * nt + med) / (nt + jnp.float32(1.0))

    @pl.when(j >= 1)
    def _():
        out_copy(j - 1).wait()

    def sub_body(r, _):
        rows = pl.ds(r * 1024, 1024)
        stage[rows, :] = buf[slot, rows, :] - new_med
        return 0

    jax.lax.fori_loop(0, n // 1024, sub_body, 0)
    out_copy(j).start()

    @pl.when(j == ng - 1)
    def _():
        out_copy(j).wait()


def kernel(x, median, num_track):
    n, c = x.shape
    grid = (c // _W,)
    nt = num_track.astype(jnp.float32).reshape(1, 1)

    fn = functools.partial(_median_shift_kernel, rank=n // 2)
    return pl.pallas_call(
        fn,
        grid=grid,
        in_specs=[
            pl.BlockSpec(memory_space=pltpu.MemorySpace.HBM),
            pl.BlockSpec((1, _W), lambda j: (0, j)),
            pl.BlockSpec(memory_space=pltpu.SMEM),
        ],
        out_specs=pl.BlockSpec(memory_space=pltpu.MemorySpace.HBM),
        out_shape=jax.ShapeDtypeStruct((n, c), jnp.float32),
        scratch_shapes=[
            pltpu.VMEM((2, n, _W), jnp.float32),
            pltpu.VMEM((n, _W), jnp.float32),
            pltpu.SemaphoreType.DMA((2,)),
            pltpu.SemaphoreType.DMA,
        ],
        compiler_params=pltpu.CompilerParams(
            dimension_semantics=("arbitrary",)),
    )(x, median, nt)
